# Initial kernel scaffold; baseline (speedup 1.0000x reference)
import jax
import jax.numpy as jnp
from jax.experimental import pallas as pl


def kernel(x):
    raise NotImplementedError("write your pallas kernel here")


# speedup vs baseline: 16.5643x; 16.5643x over previous
import functools

import jax
import jax.numpy as jnp
from jax import lax
from jax.experimental import pallas as pl
from jax.experimental.pallas import tpu as pltpu

N_ROWS = 96
ROW = 512 * 512
K_BOT = 5243
K_TOP = 256900


def _monotone_u32(f):
    i = lax.bitcast_convert_type(f, jnp.int32)
    flip = lax.shift_right_arithmetic(i, 31) & jnp.int32(0x7FFFFFFF)
    k = i ^ flip
    return lax.bitcast_convert_type(k, jnp.uint32) + jnp.uint32(0x80000000)


def _u32_to_f32(u):
    i = lax.bitcast_convert_type(u + jnp.uint32(0x80000000), jnp.int32)
    flip = lax.shift_right_arithmetic(i, 31) & jnp.int32(0x7FFFFFFF)
    return lax.bitcast_convert_type(i ^ flip, jnp.float32)


def _select_body(x_ref, stats_ref):
    xb = x_ref[0]
    ukey = _monotone_u32(xb)

    def bit_step(b, carry):
        p_lo, p_hi = carry
        bit = jnp.uint32(1) << (jnp.uint32(31) - b.astype(jnp.uint32))
        low = bit - jnp.uint32(1)
        t_lo = p_lo | low
        t_hi = p_hi | low
        c_lo = jnp.sum((ukey <= t_lo).astype(jnp.int32))
        c_hi = jnp.sum((ukey <= t_hi).astype(jnp.int32))
        p_lo = jnp.where(c_lo >= K_BOT + 1, p_lo, p_lo | bit)
        p_hi = jnp.where(c_hi >= K_TOP + 1, p_hi, p_hi | bit)
        return p_lo, p_hi

    p_lo, p_hi = lax.fori_loop(0, 32, bit_step, (jnp.uint32(0), jnp.uint32(0)))
    bot = _u32_to_f32(p_lo)
    top = _u32_to_f32(p_hi)
    mn = jnp.min(xb)
    mx = jnp.max(xb)
    col = lax.broadcasted_iota(jnp.int32, (1, 1, 128), 2)
    vec = jnp.where(col == 0, bot,
          jnp.where(col == 1, top,
          jnp.where(col == 2, mn,
          jnp.where(col == 3, mx, 0.0))))
    stats_ref[...] = vec


def _normalize_body(stats_all_ref, x_ref, stats_row_ref, out_ref):
    s = stats_all_ref[:, 0, :]
    bot_raw, top_raw = s[:, 0], s[:, 1]
    mn, mx = s[:, 2], s[:, 3]
    same = top_raw == bot_raw
    top1 = jnp.where(same, mx, top_raw)
    bot1 = jnp.where(same, mn, bot_raw)
    all_black = jnp.any(top1 == 0.0)
    all_const = jnp.any(top1 == bot1)
    b_row = stats_row_ref[0, 0, 0]
    t_row = stats_row_ref[0, 0, 1]
    same_r = t_row == b_row
    t1 = jnp.where(same_r, stats_row_ref[0, 0, 3], t_row)
    b1 = jnp.where(same_r, stats_row_ref[0, 0, 2], b_row)
    t = jnp.where(all_black, jnp.float32(1.0), t1)
    b = jnp.where(jnp.logical_and(jnp.logical_not(all_black), all_const),
                  jnp.float32(0.0), b1)
    scale = jnp.float32(1.0) / (t - b)
    out_ref[...] = jnp.clip((x_ref[...] - b) * scale, 0.0, 1.0)


def kernel(x):
    stats = pl.pallas_call(
        _select_body,
        grid=(N_ROWS,),
        in_specs=[pl.BlockSpec((1, 512, 512), lambda i: (i, 0, 0))],
        out_specs=pl.BlockSpec((1, 1, 128), lambda i: (i, 0, 0)),
        out_shape=jax.ShapeDtypeStruct((N_ROWS, 1, 128), jnp.float32),
    )(x)

    out = pl.pallas_call(
        _normalize_body,
        grid=(N_ROWS,),
        in_specs=[
            pl.BlockSpec((N_ROWS, 1, 128), lambda i: (0, 0, 0)),
            pl.BlockSpec((1, 512, 512), lambda i: (i, 0, 0)),
            pl.BlockSpec((1, 1, 128), lambda i: (i, 0, 0)),
        ],
        out_specs=pl.BlockSpec((1, 512, 512), lambda i: (i, 0, 0)),
        out_shape=jax.ShapeDtypeStruct((N_ROWS, 512, 512), jnp.float32),
    )(stats, x, stats)
    return out
